# baseline (speedup 1.0000x reference)
import functools

import jax
import jax.numpy as jnp
from jax import lax
from jax.experimental import pallas as pl
from jax.experimental.pallas import tpu as pltpu
from jax.experimental.pallas import tpu_sc as plsc

L = 16
NC, NS = 2, 16
NW = NC * NS

EMB = 64
KW = EMB // 2
NNEG = 200
SLOT = 208
NG = SLOT // L
CB = 4
ROWS = CB * SLOT
IDXBLK = 128


def _mf_kernel(nb, chunks,
               user_id, pos_items, neg_items, user_table, item_table,
               pos_out, neg_out,
               uid_v, pid_v, idx_d, rows_d, q_d, nbuf_d, pos_acc,
               sem_i, sem_r, sem_q, sem_o):
    wid = lax.axis_index("s") * NC + lax.axis_index("c")
    base = wid * nb
    lane = lax.broadcasted_iota(jnp.int32, (L,), 0)

    pltpu.sync_copy(user_id.at[pl.ds(base, nb)], uid_v)
    pltpu.sync_copy(pos_items.at[pl.ds(base, nb)], pid_v)

    def ids_descs(c, idx, p):
        cb0 = base + c * CB
        return [pltpu.make_async_copy(
            neg_items.at[pl.ds(pl.multiple_of((cb0 + b) * NNEG, 8), NNEG)],
            idx.at[pl.ds(b * SLOT, NNEG)], sem_i[p]) for b in range(CB)]

    def rows_descs(idx, rows):
        ds = []
        nfull, rem = ROWS // IDXBLK, ROWS % IDXBLK
        for j in range(nfull):
            ds.append(pltpu.make_async_copy(
                item_table.at[idx.at[pl.ds(j * IDXBLK, IDXBLK)]],
                rows.at[pl.ds(j * IDXBLK, IDXBLK)], sem_r))
        if rem:
            ds.append(pltpu.make_async_copy(
                item_table.at[idx.at[pl.ds(nfull * IDXBLK, rem)]],
                rows.at[pl.ds(nfull * IDXBLK, rem)], sem_r))
        return ds

    def q_desc(c, qb, p):
        qidx = plsc.load_gather(uid_v, [jnp.minimum(c * CB + lane, nb - 1)])
        return pltpu.make_async_copy(user_table.at[qidx], qb, sem_q[p])

    def scatter_pos(c, idx):
        pvals = plsc.load_gather(pid_v, [jnp.minimum(c * CB + lane, nb - 1)])
        plsc.store_scatter(idx, [lane * SLOT + NNEG], pvals, mask=lane < CB)

    def out_descs(c, nbuf, p):
        cb0 = base + c * CB
        return [pltpu.make_async_copy(
            nbuf.at[pl.ds(b * SLOT, NNEG)],
            neg_out.at[pl.ds(pl.multiple_of((cb0 + b) * NNEG, 8), NNEG)],
            sem_o[p]) for b in range(CB)]

    def compute(c, rows, qb, nbuf):
        for b in range(CB):
            qrow = jnp.full((L,), b, jnp.int32)
            row0 = b * SLOT

            def dbody(k, accs):
                col = (k + lane) & (KW - 1)
                qw = plsc.load_gather(qb, [qrow, col])
                qa, qb2 = plsc.unpack(plsc.bitcast(qw, jnp.bfloat16),
                                      format=plsc.PackFormat.INTERLEAVED)
                out = []
                for g in range(NG):
                    rw = plsc.load_gather(rows, [row0 + g * L + lane, col])
                    ra, rb = plsc.unpack(plsc.bitcast(rw, jnp.bfloat16),
                                         format=plsc.PackFormat.INTERLEAVED)
                    out.append(accs[g] + ra * qa + rb * qb2)
                return tuple(out)

            accs = lax.fori_loop(
                0, KW, dbody,
                tuple(jnp.zeros((L,), jnp.float32) for _ in range(NG)))
            for g in range(NG):
                nbuf[pl.ds(row0 + g * L, L)] = accs[g]
        pv = plsc.load_gather(
            nbuf, [jnp.minimum(lane * SLOT + NNEG, ROWS - 1)])
        plsc.store_scatter(pos_acc, [c * CB + lane], pv, mask=lane < CB)

    zeros_i = jnp.zeros((L,), jnp.int32)
    for p in range(2):
        for k in range(ROWS // L):
            idx_d[p][pl.ds(k * L, L)] = zeros_i

    for d in ids_descs(0, idx_d[0], 0):
        d.start()
    for d in ids_descs(0, idx_d[0], 0):
        d.wait()
    scatter_pos(0, idx_d[0])
    for d in rows_descs(idx_d[0], rows_d[0]):
        d.start()
    q_desc(0, q_d[0], 0).start()
    for d in ids_descs(1, idx_d[1], 1):
        d.start()

    def pair_body(cc, carry):
        for ph in range(2):
            c = 2 * cc + ph
            idx, rows, qb, nbuf = (
                idx_d[ph], rows_d[ph], q_d[ph], nbuf_d[ph])
            idxn, rowsn, qbn = (
                idx_d[1 - ph], rows_d[1 - ph], q_d[1 - ph])

            @pl.when(c + 1 < chunks)
            def _():
                for d in ids_descs(c + 1, idxn, 1 - ph):
                    d.wait()
                scatter_pos(c + 1, idxn)
                for d in rows_descs(idxn, rowsn):
                    d.start()
                q_desc(c + 1, qbn, 1 - ph).start()

            for d in rows_descs(idx, rows):
                d.wait()
            q_desc(c, qb, ph).wait()

            @pl.when(c + 2 < chunks)
            def _():
                for d in ids_descs(c + 2, idx, ph):
                    d.start()

            @pl.when(c >= 2)
            def _():
                for d in out_descs(c, nbuf, ph):
                    d.wait()

            compute(c, rows, qb, nbuf)
            for d in out_descs(c, nbuf, ph):
                d.start()
        return carry

    lax.fori_loop(0, chunks // 2, pair_body, 0)

    for d in out_descs(chunks - 2, nbuf_d[0], 0):
        d.wait()
    for d in out_descs(chunks - 1, nbuf_d[1], 1):
        d.wait()
    pltpu.sync_copy(pos_acc, pos_out.at[pl.ds(base, nb)])


def _pack_bf16(table):
    bb = table.astype(jnp.bfloat16).reshape(table.shape[0], KW, 2)
    return jax.lax.bitcast_convert_type(bb, jnp.int32)


def kernel(user_id, pos_items, neg_items, user_table, item_table):
    bsz = user_id.shape[0]
    n_neg = neg_items.shape[1]
    assert n_neg == NNEG and user_table.shape[1] == EMB
    assert bsz % (NW * 2 * CB) == 0
    nb = bsz // NW
    chunks = nb // CB

    mesh = plsc.VectorSubcoreMesh(
        core_axis_name="c", subcore_axis_name="s",
        num_cores=NC, num_subcores=NS)
    k = pl.kernel(
        functools.partial(_mf_kernel, nb, chunks),
        out_type=(
            jax.ShapeDtypeStruct((bsz,), jnp.float32),
            jax.ShapeDtypeStruct((bsz * n_neg,), jnp.float32),
        ),
        mesh=mesh,
        compiler_params=pltpu.CompilerParams(
            needs_layout_passes=False, use_tc_tiling_on_sc=False),
        scratch_types=[
            pltpu.VMEM((nb,), jnp.int32),
            pltpu.VMEM((nb,), jnp.int32),
            [pltpu.VMEM((ROWS,), jnp.int32)] * 2,
            [pltpu.VMEM((ROWS, KW), jnp.int32)] * 2,
            [pltpu.VMEM((L, KW), jnp.int32)] * 2,
            [pltpu.VMEM((ROWS,), jnp.float32)] * 2,
            pltpu.VMEM((nb,), jnp.float32),
            [pltpu.SemaphoreType.DMA] * 2,
            pltpu.SemaphoreType.DMA,
            [pltpu.SemaphoreType.DMA] * 2,
            [pltpu.SemaphoreType.DMA] * 2,
        ],
        name="mf_scores_sc",
    )
    pos_score, neg_flat = k(
        user_id.astype(jnp.int32),
        pos_items.astype(jnp.int32),
        neg_items.reshape(-1).astype(jnp.int32),
        _pack_bf16(user_table),
        _pack_bf16(item_table),
    )
    return pos_score, neg_flat.reshape(bsz, n_neg)


# speedup vs baseline: 1.4218x; 1.4218x over previous
import functools

import jax
import jax.numpy as jnp
from jax import lax
from jax.experimental import pallas as pl
from jax.experimental.pallas import tpu as pltpu
from jax.experimental.pallas import tpu_sc as plsc

L = 16
NC, NS = 2, 16
NW = NC * NS

EMB = 64
KW = EMB // 2
NNEG = 200
SLOT = 208
NG = SLOT // L
CB = 4
ROWS = CB * SLOT
IDXBLK = 128


def _mf_kernel(nb, chunks,
               user_id, pos_items, neg_items, user_table, item_table,
               pos_out, neg_out,
               uid_v, pid_v, idx_d, rows_d, q_d, nbuf_d, pos_acc,
               sem_i, sem_r, sem_q, sem_o):
    wid = lax.axis_index("s") * NC + lax.axis_index("c")
    base = wid * nb
    lane = lax.broadcasted_iota(jnp.int32, (L,), 0)

    pltpu.sync_copy(user_id.at[pl.ds(base, nb)], uid_v)
    pltpu.sync_copy(pos_items.at[pl.ds(base, nb)], pid_v)

    def ids_descs(c, idx, p):
        cb0 = base + c * CB
        return [pltpu.make_async_copy(
            neg_items.at[pl.ds(pl.multiple_of((cb0 + b) * NNEG, 8), NNEG)],
            idx.at[pl.ds(b * SLOT, NNEG)], sem_i[p]) for b in range(CB)]

    def rows_descs(idx, rows):
        ds = []
        nfull, rem = ROWS // IDXBLK, ROWS % IDXBLK
        for j in range(nfull):
            ds.append(pltpu.make_async_copy(
                item_table.at[idx.at[pl.ds(j * IDXBLK, IDXBLK)]],
                rows.at[pl.ds(j * IDXBLK, IDXBLK)], sem_r))
        if rem:
            ds.append(pltpu.make_async_copy(
                item_table.at[idx.at[pl.ds(nfull * IDXBLK, rem)]],
                rows.at[pl.ds(nfull * IDXBLK, rem)], sem_r))
        return ds

    def q_desc(c, qb, p):
        qidx = plsc.load_gather(uid_v, [jnp.minimum(c * CB + lane, nb - 1)])
        return pltpu.make_async_copy(user_table.at[qidx], qb, sem_q[p])

    def scatter_pos(c, idx):
        pvals = plsc.load_gather(pid_v, [jnp.minimum(c * CB + lane, nb - 1)])
        plsc.store_scatter(idx, [lane * SLOT + NNEG], pvals, mask=lane < CB)

    def out_descs(c, nbuf, p):
        cb0 = base + c * CB
        return [pltpu.make_async_copy(
            nbuf.at[pl.ds(b * SLOT, NNEG)],
            neg_out.at[pl.ds(pl.multiple_of((cb0 + b) * NNEG, 8), NNEG)],
            sem_o[p]) for b in range(CB)]

    def compute(c, rows, qb, nbuf):
        for b in range(CB):
            qrow = jnp.full((L,), b, jnp.int32)
            row0 = b * SLOT

            def dbody(k, accs):
                col = (k + lane) & (KW - 1)
                qw = plsc.load_gather(qb, [qrow, col])
                qa, qb2 = plsc.unpack(plsc.bitcast(qw, jnp.bfloat16),
                                      format=plsc.PackFormat.INTERLEAVED)
                out = []
                for g in range(NG):
                    rw = plsc.load_gather(rows, [row0 + g * L + lane, col])
                    ra, rb = plsc.unpack(plsc.bitcast(rw, jnp.bfloat16),
                                         format=plsc.PackFormat.INTERLEAVED)
                    out.append(accs[g] + ra * qa + rb * qb2)
                return tuple(out)

            accs = lax.fori_loop(
                0, KW, dbody,
                tuple(jnp.zeros((L,), jnp.float32) for _ in range(NG)))
            for g in range(NG):
                nbuf[pl.ds(row0 + g * L, L)] = accs[g]
        pv = plsc.load_gather(
            nbuf, [jnp.minimum(lane * SLOT + NNEG, ROWS - 1)])
        plsc.store_scatter(pos_acc, [c * CB + lane], pv, mask=lane < CB)

    zeros_i = jnp.zeros((L,), jnp.int32)
    for p in range(2):
        for k in range(ROWS // L):
            idx_d[p][pl.ds(k * L, L)] = zeros_i

    for d in ids_descs(0, idx_d[0], 0):
        d.start()
    for d in ids_descs(0, idx_d[0], 0):
        d.wait()
    scatter_pos(0, idx_d[0])
    for d in rows_descs(idx_d[0], rows_d[0]):
        d.start()
    q_desc(0, q_d[0], 0).start()
    for d in ids_descs(1, idx_d[1], 1):
        d.start()

    def pair_body(cc, carry):
        for ph in range(2):
            c = 2 * cc + ph
            idx, rows, qb, nbuf = (
                idx_d[ph], rows_d[ph], q_d[ph], nbuf_d[ph])
            idxn, rowsn, qbn = (
                idx_d[1 - ph], rows_d[1 - ph], q_d[1 - ph])

            @pl.when(c + 1 < chunks)
            def _():
                for d in ids_descs(c + 1, idxn, 1 - ph):
                    d.wait()
                scatter_pos(c + 1, idxn)
                for d in rows_descs(idxn, rowsn):
                    d.start()
                q_desc(c + 1, qbn, 1 - ph).start()

            for d in rows_descs(idx, rows):
                d.wait()
            q_desc(c, qb, ph).wait()

            @pl.when(c + 2 < chunks)
            def _():
                for d in ids_descs(c + 2, idx, ph):
                    d.start()

            @pl.when(c >= 2)
            def _():
                for d in out_descs(c, nbuf, ph):
                    d.wait()

            compute(c, rows, qb, nbuf)
            for d in out_descs(c, nbuf, ph):
                d.start()
        return carry

    lax.fori_loop(0, chunks // 2, pair_body, 0)

    for d in out_descs(chunks - 2, nbuf_d[0], 0):
        d.wait()
    for d in out_descs(chunks - 1, nbuf_d[1], 1):
        d.wait()
    pltpu.sync_copy(pos_acc, pos_out.at[pl.ds(base, nb)])


def _pack_bf16(table):
    lo = jax.lax.bitcast_convert_type(
        table[:, :KW].astype(jnp.bfloat16), jnp.uint16).astype(jnp.int32)
    hi = jax.lax.bitcast_convert_type(
        table[:, KW:].astype(jnp.bfloat16), jnp.uint16).astype(jnp.int32)
    return lo | (hi << 16)


def kernel(user_id, pos_items, neg_items, user_table, item_table):
    bsz = user_id.shape[0]
    n_neg = neg_items.shape[1]
    assert n_neg == NNEG and user_table.shape[1] == EMB
    assert bsz % (NW * 2 * CB) == 0
    nb = bsz // NW
    chunks = nb // CB

    mesh = plsc.VectorSubcoreMesh(
        core_axis_name="c", subcore_axis_name="s",
        num_cores=NC, num_subcores=NS)
    k = pl.kernel(
        functools.partial(_mf_kernel, nb, chunks),
        out_type=(
            jax.ShapeDtypeStruct((bsz,), jnp.float32),
            jax.ShapeDtypeStruct((bsz * n_neg,), jnp.float32),
        ),
        mesh=mesh,
        compiler_params=pltpu.CompilerParams(
            needs_layout_passes=False, use_tc_tiling_on_sc=False),
        scratch_types=[
            pltpu.VMEM((nb,), jnp.int32),
            pltpu.VMEM((nb,), jnp.int32),
            [pltpu.VMEM((ROWS,), jnp.int32)] * 2,
            [pltpu.VMEM((ROWS, KW), jnp.int32)] * 2,
            [pltpu.VMEM((L, KW), jnp.int32)] * 2,
            [pltpu.VMEM((ROWS,), jnp.float32)] * 2,
            pltpu.VMEM((nb,), jnp.float32),
            [pltpu.SemaphoreType.DMA] * 2,
            pltpu.SemaphoreType.DMA,
            [pltpu.SemaphoreType.DMA] * 2,
            [pltpu.SemaphoreType.DMA] * 2,
        ],
        name="mf_scores_sc",
    )
    pos_score, neg_flat = k(
        user_id.astype(jnp.int32),
        pos_items.astype(jnp.int32),
        neg_items.reshape(-1).astype(jnp.int32),
        _pack_bf16(user_table),
        _pack_bf16(item_table),
    )
    return pos_score, neg_flat.reshape(bsz, n_neg)


# speedup vs baseline: 1.6311x; 1.1472x over previous
import functools

import jax
import jax.numpy as jnp
from jax import lax
from jax.experimental import pallas as pl
from jax.experimental.pallas import tpu as pltpu
from jax.experimental.pallas import tpu_sc as plsc

L = 16
NC, NS = 2, 16
NW = NC * NS

EMB = 64
KW = EMB // 2
NNEG = 200
SLOT = 208
NG = SLOT // L
CB = 4
ROWS = CB * SLOT
IDXBLK = 128


def _mf_kernel(nb, chunks,
               user_id, pos_items, neg_items, user_table, item_table,
               pos_out, neg_out,
               uid_v, pid_v, idx_d, rows_d, q_d, nbuf_d, pos_acc,
               sem_i, sem_r, sem_q, sem_o):
    wid = lax.axis_index("s") * NC + lax.axis_index("c")
    base = wid * nb
    lane = lax.broadcasted_iota(jnp.int32, (L,), 0)

    pltpu.sync_copy(user_id.at[pl.ds(base, nb)], uid_v)
    pltpu.sync_copy(pos_items.at[pl.ds(base, nb)], pid_v)

    def ids_descs(c, idx, p):
        cb0 = base + c * CB
        return [pltpu.make_async_copy(
            neg_items.at[pl.ds(pl.multiple_of((cb0 + b) * NNEG, 8), NNEG)],
            idx.at[pl.ds(b * SLOT, NNEG)], sem_i[p]) for b in range(CB)]

    def rows_descs(idx, rows):
        ds = []
        nfull, rem = ROWS // IDXBLK, ROWS % IDXBLK
        for j in range(nfull):
            ds.append(pltpu.make_async_copy(
                item_table.at[idx.at[pl.ds(j * IDXBLK, IDXBLK)]],
                rows.at[pl.ds(j * IDXBLK, IDXBLK)], sem_r))
        if rem:
            ds.append(pltpu.make_async_copy(
                item_table.at[idx.at[pl.ds(nfull * IDXBLK, rem)]],
                rows.at[pl.ds(nfull * IDXBLK, rem)], sem_r))
        return ds

    def q_desc(c, qb, p):
        qidx = plsc.load_gather(uid_v, [jnp.minimum(c * CB + lane, nb - 1)])
        return pltpu.make_async_copy(user_table.at[qidx], qb, sem_q[p])

    def scatter_pos(c, idx):
        pvals = plsc.load_gather(pid_v, [jnp.minimum(c * CB + lane, nb - 1)])
        plsc.store_scatter(idx, [lane * SLOT + NNEG], pvals, mask=lane < CB)

    def out_descs(c, nbuf, p):
        cb0 = base + c * CB
        return [pltpu.make_async_copy(
            nbuf.at[pl.ds(b * SLOT, NNEG)],
            neg_out.at[pl.ds(pl.multiple_of((cb0 + b) * NNEG, 8), NNEG)],
            sem_o[p]) for b in range(CB)]

    def compute(c, rows, qb, nbuf):
        for b in range(CB):
            qrow = jnp.full((L,), b, jnp.int32)
            row0 = b * SLOT

            def dbody(k, accs):
                col = (k + lane) & (KW - 1)
                qa = plsc.load_gather(qb, [qrow, col])
                qb2 = plsc.load_gather(qb, [qrow, col + KW])
                out = []
                for g in range(NG):
                    rw = plsc.load_gather(rows, [row0 + g * L + lane, col])
                    ra, rb = plsc.unpack(plsc.bitcast(rw, jnp.bfloat16),
                                         format=plsc.PackFormat.INTERLEAVED)
                    out.append(accs[g] + ra * qa + rb * qb2)
                return tuple(out)

            accs = lax.fori_loop(
                0, KW, dbody,
                tuple(jnp.zeros((L,), jnp.float32) for _ in range(NG)))
            for g in range(NG):
                nbuf[pl.ds(row0 + g * L, L)] = accs[g]
        pv = plsc.load_gather(
            nbuf, [jnp.minimum(lane * SLOT + NNEG, ROWS - 1)])
        plsc.store_scatter(pos_acc, [c * CB + lane], pv, mask=lane < CB)

    zeros_i = jnp.zeros((L,), jnp.int32)
    for p in range(2):
        for k in range(ROWS // L):
            idx_d[p][pl.ds(k * L, L)] = zeros_i

    for d in ids_descs(0, idx_d[0], 0):
        d.start()
    for d in ids_descs(0, idx_d[0], 0):
        d.wait()
    scatter_pos(0, idx_d[0])
    for d in rows_descs(idx_d[0], rows_d[0]):
        d.start()
    q_desc(0, q_d[0], 0).start()
    for d in ids_descs(1, idx_d[1], 1):
        d.start()

    def pair_body(cc, carry):
        for ph in range(2):
            c = 2 * cc + ph
            idx, rows, qb, nbuf = (
                idx_d[ph], rows_d[ph], q_d[ph], nbuf_d[ph])
            idxn, rowsn, qbn = (
                idx_d[1 - ph], rows_d[1 - ph], q_d[1 - ph])

            @pl.when(c + 1 < chunks)
            def _():
                for d in ids_descs(c + 1, idxn, 1 - ph):
                    d.wait()
                scatter_pos(c + 1, idxn)
                for d in rows_descs(idxn, rowsn):
                    d.start()
                q_desc(c + 1, qbn, 1 - ph).start()

            for d in rows_descs(idx, rows):
                d.wait()
            q_desc(c, qb, ph).wait()

            @pl.when(c + 2 < chunks)
            def _():
                for d in ids_descs(c + 2, idx, ph):
                    d.start()

            @pl.when(c >= 2)
            def _():
                for d in out_descs(c, nbuf, ph):
                    d.wait()

            compute(c, rows, qb, nbuf)
            for d in out_descs(c, nbuf, ph):
                d.start()
        return carry

    lax.fori_loop(0, chunks // 2, pair_body, 0)

    for d in out_descs(chunks - 2, nbuf_d[0], 0):
        d.wait()
    for d in out_descs(chunks - 1, nbuf_d[1], 1):
        d.wait()
    pltpu.sync_copy(pos_acc, pos_out.at[pl.ds(base, nb)])


def _pack_bf16(table):
    lo = jax.lax.bitcast_convert_type(
        table[:, :KW].astype(jnp.bfloat16), jnp.uint16).astype(jnp.int32)
    hi = jax.lax.bitcast_convert_type(
        table[:, KW:].astype(jnp.bfloat16), jnp.uint16).astype(jnp.int32)
    return lo | (hi << 16)


def kernel(user_id, pos_items, neg_items, user_table, item_table):
    bsz = user_id.shape[0]
    n_neg = neg_items.shape[1]
    assert n_neg == NNEG and user_table.shape[1] == EMB
    assert bsz % (NW * 2 * CB) == 0
    nb = bsz // NW
    chunks = nb // CB

    mesh = plsc.VectorSubcoreMesh(
        core_axis_name="c", subcore_axis_name="s",
        num_cores=NC, num_subcores=NS)
    k = pl.kernel(
        functools.partial(_mf_kernel, nb, chunks),
        out_type=(
            jax.ShapeDtypeStruct((bsz,), jnp.float32),
            jax.ShapeDtypeStruct((bsz * n_neg,), jnp.float32),
        ),
        mesh=mesh,
        compiler_params=pltpu.CompilerParams(
            needs_layout_passes=False, use_tc_tiling_on_sc=False),
        scratch_types=[
            pltpu.VMEM((nb,), jnp.int32),
            pltpu.VMEM((nb,), jnp.int32),
            [pltpu.VMEM((ROWS,), jnp.int32)] * 2,
            [pltpu.VMEM((ROWS, KW), jnp.int32)] * 2,
            [pltpu.VMEM((L, EMB), jnp.float32)] * 2,
            [pltpu.VMEM((ROWS,), jnp.float32)] * 2,
            pltpu.VMEM((nb,), jnp.float32),
            [pltpu.SemaphoreType.DMA] * 2,
            pltpu.SemaphoreType.DMA,
            [pltpu.SemaphoreType.DMA] * 2,
            [pltpu.SemaphoreType.DMA] * 2,
        ],
        name="mf_scores_sc",
    )
    pos_score, neg_flat = k(
        user_id.astype(jnp.int32),
        pos_items.astype(jnp.int32),
        neg_items.reshape(-1).astype(jnp.int32),
        user_table,
        _pack_bf16(item_table),
    )
    return pos_score, neg_flat.reshape(bsz, n_neg)
